# baseline (speedup 1.0000x reference)
import functools

import jax
import jax.numpy as jnp
from jax import lax
from jax.experimental import pallas as pl
from jax.experimental.pallas import tpu as pltpu
from jax.experimental.pallas import tpu_sc as plsc

NC = 2
NS = 16
NW = NC * NS
LANES = 16
CHUNK = 64


def _make_sc_kernel(n_nodes, d_model, e_pad):
    e_tile = e_pad // NW
    n_chunks = e_tile // CHUNK
    n_dblk = d_model // LANES

    mesh = plsc.VectorSubcoreMesh(core_axis_name="c", subcore_axis_name="s")

    @functools.partial(
        pl.kernel,
        out_type=jax.ShapeDtypeStruct((e_pad,), jnp.float32),
        mesh=mesh,
        compiler_params=pltpu.CompilerParams(
            use_tc_tiling_on_sc=False, needs_layout_passes=False),
        scratch_types=[
            pltpu.VMEM((e_tile,), jnp.int32),
            pltpu.VMEM((e_tile,), jnp.int32),
            pltpu.VMEM((e_tile,), jnp.float32),
            pltpu.VMEM((CHUNK, d_model), jnp.float32),
            pltpu.VMEM((CHUNK, d_model), jnp.float32),
            pltpu.SemaphoreType.DMA,
        ],
    )
    def sc_kernel(x_hbm, src_hbm, dst_hbm, out_hbm,
                  src_v, dst_v, out_v, bu, bv, sem):
        wid = lax.axis_index("s") * NC + lax.axis_index("c")
        base = pl.multiple_of(wid * e_tile, 8)

        pltpu.sync_copy(src_hbm.at[pl.ds(base, e_tile)], src_v)
        pltpu.sync_copy(dst_hbm.at[pl.ds(base, e_tile)], dst_v)

        def chunk_body(c, carry):
            cb = pl.multiple_of(c * CHUNK, 8)
            cu = pltpu.async_copy(x_hbm.at[src_v.at[pl.ds(cb, CHUNK)]], bu, sem)
            cv = pltpu.async_copy(x_hbm.at[dst_v.at[pl.ds(cb, CHUNK)]], bv, sem)
            cu.wait()
            cv.wait()

            lane = lax.iota(jnp.int32, LANES)

            def grp_body(g, carry2):
                e_vec = g * LANES + lane

                def feat_body(d, acc):
                    d_vec = jnp.full((LANES,), d, jnp.int32)
                    hu = plsc.load_gather(bu, [e_vec, d_vec])
                    hv = plsc.load_gather(bv, [e_vec, d_vec])
                    return acc + hu * hv

                scores = lax.fori_loop(0, d_model, feat_body,
                                       jnp.zeros((LANES,), jnp.float32),
                                       unroll=8)
                out_v[pl.ds(pl.multiple_of(cb + g * LANES, 8), LANES)] = scores
                return carry2

            lax.fori_loop(0, CHUNK // LANES, grp_body, 0, unroll=False)
            return carry

        lax.fori_loop(0, n_chunks, chunk_body, 0, unroll=False)
        pltpu.sync_copy(out_v, out_hbm.at[pl.ds(base, e_tile)])

    return sc_kernel


def kernel(x, edge_index):
    n_nodes, d_model = x.shape
    n_edges = edge_index.shape[1]
    e_pad = ((n_edges + NW * CHUNK - 1) // (NW * CHUNK)) * (NW * CHUNK)

    src = edge_index[0].astype(jnp.int32)
    dst = edge_index[1].astype(jnp.int32)
    if e_pad != n_edges:
        pad = e_pad - n_edges
        src = jnp.concatenate([src, jnp.zeros((pad,), jnp.int32)])
        dst = jnp.concatenate([dst, jnp.zeros((pad,), jnp.int32)])

    score = _make_sc_kernel(n_nodes, d_model, e_pad)(x, src, dst)
    return score[:n_edges].reshape(n_edges, 1)


# speedup vs baseline: 2.8159x; 2.8159x over previous
import functools

import jax
import jax.numpy as jnp
from jax import lax
from jax.experimental import pallas as pl
from jax.experimental.pallas import tpu as pltpu
from jax.experimental.pallas import tpu_sc as plsc

NC = 2
NS = 16
NW = NC * NS
LANES = 16
CHUNK = 64


def _make_sc_kernel(n_nodes, d_model, e_pad):
    e_tile = e_pad // NW
    n_chunks = e_tile // CHUNK
    n_dblk = d_model // LANES
    assert n_chunks % 2 == 0

    mesh = plsc.VectorSubcoreMesh(core_axis_name="c", subcore_axis_name="s")

    @functools.partial(
        pl.kernel,
        out_type=jax.ShapeDtypeStruct((e_pad,), jnp.float32),
        mesh=mesh,
        compiler_params=pltpu.CompilerParams(
            use_tc_tiling_on_sc=False, needs_layout_passes=False),
        scratch_types=[
            pltpu.VMEM((e_tile,), jnp.int32),
            pltpu.VMEM((e_tile,), jnp.int32),
            pltpu.VMEM((e_tile,), jnp.float32),
            pltpu.VMEM((CHUNK, d_model), jnp.float32),
            pltpu.VMEM((CHUNK, d_model), jnp.float32),
            pltpu.VMEM((CHUNK, d_model), jnp.float32),
            pltpu.VMEM((CHUNK, d_model), jnp.float32),
            pltpu.SemaphoreType.DMA,
            pltpu.SemaphoreType.DMA,
        ],
    )
    def sc_kernel(x_hbm, src_hbm, dst_hbm, out_hbm,
                  src_v, dst_v, out_v, bu0, bv0, bu1, bv1, sem0, sem1):
        wid = lax.axis_index("s") * NC + lax.axis_index("c")
        base = pl.multiple_of(wid * e_tile, 8)

        pltpu.sync_copy(src_hbm.at[pl.ds(base, e_tile)], src_v)
        pltpu.sync_copy(dst_hbm.at[pl.ds(base, e_tile)], dst_v)

        def fire(c, bu, bv, sem):
            cb = pl.multiple_of(c * CHUNK, 8)
            pltpu.async_copy(x_hbm.at[src_v.at[pl.ds(cb, CHUNK)]], bu, sem)
            pltpu.async_copy(x_hbm.at[dst_v.at[pl.ds(cb, CHUNK)]], bv, sem)

        def drain(bu, bv, sem):
            pltpu.make_async_copy(x_hbm.at[src_v.at[pl.ds(0, CHUNK)]], bu, sem).wait()
            pltpu.make_async_copy(x_hbm.at[dst_v.at[pl.ds(0, CHUNK)]], bv, sem).wait()

        lane = lax.iota(jnp.int32, LANES)

        def compute(c, bu, bv):
            cb = c * CHUNK

            def grp_body(g, carry2):
                gb = g * LANES
                vec = jnp.zeros((LANES,), jnp.float32)
                for j in range(LANES):
                    e = gb + j
                    acc = bu[e, pl.ds(0, LANES)] * bv[e, pl.ds(0, LANES)]
                    for d in range(1, n_dblk):
                        acc = acc + (bu[e, pl.ds(d * LANES, LANES)]
                                     * bv[e, pl.ds(d * LANES, LANES)])
                    vec = jnp.where(lane == j, jnp.sum(acc), vec)
                out_v[pl.ds(pl.multiple_of(cb + gb, 8), LANES)] = vec
                return carry2

            lax.fori_loop(0, CHUNK // LANES, grp_body, 0, unroll=False)

        fire(0, bu0, bv0, sem0)
        fire(1, bu1, bv1, sem1)

        def pair_body(p, carry):
            c0 = 2 * p
            drain(bu0, bv0, sem0)

            @pl.when(p + 1 < n_chunks // 2)
            def _():
                fire(c0 + 2, bu0, bv0, sem0)

            compute(c0, bu0, bv0)
            drain(bu1, bv1, sem1)

            @pl.when(p + 1 < n_chunks // 2)
            def _():
                fire(c0 + 3, bu1, bv1, sem1)

            compute(c0 + 1, bu1, bv1)
            return carry

        lax.fori_loop(0, n_chunks // 2, pair_body, 0, unroll=False)
        pltpu.sync_copy(out_v, out_hbm.at[pl.ds(base, e_tile)])

    return sc_kernel


def kernel(x, edge_index):
    n_nodes, d_model = x.shape
    n_edges = edge_index.shape[1]
    grain = NW * CHUNK * 2
    e_pad = ((n_edges + grain - 1) // grain) * grain

    src = edge_index[0].astype(jnp.int32)
    dst = edge_index[1].astype(jnp.int32)
    if e_pad != n_edges:
        pad = e_pad - n_edges
        src = jnp.concatenate([src, jnp.zeros((pad,), jnp.int32)])
        dst = jnp.concatenate([dst, jnp.zeros((pad,), jnp.int32)])

    score = _make_sc_kernel(n_nodes, d_model, e_pad)(x, src, dst)
    return score[:n_edges].reshape(n_edges, 1)
